# baseline (speedup 1.0000x reference)
import jax
import jax.numpy as jnp
from jax.experimental import pallas as pl
from jax.experimental.pallas import tpu as pltpu

BETA = 0.25
TILE = 1024
NCHAIN = 8
_F32 = jnp.float32
_BF16 = jnp.bfloat16


def _bdot(a, b):
    return jnp.dot(a.astype(_BF16), b, preferred_element_type=_F32)


def _rqvae_tile(x_ref, ew0, eb0, ew1, eb1, ew2, eb2,
                dw0, db0, dw1, db1, dw2, db2,
                dr0, dr1, dr2, er0, er1, er2, out_ref):
    x = x_ref[...]
    t = x.shape[0]
    c = t // NCHAIN
    xs = [x[i * c:(i + 1) * c] for i in range(NCHAIN)]
    hs = [jnp.maximum(_bdot(v, ew0[...]) + eb0[0, :], 0.0) for v in xs]
    hs = [jnp.maximum(_bdot(v, ew1[...]) + eb1[0, :], 0.0) for v in hs]
    res0s = [_bdot(v, ew2[...]) + eb2[0, :] for v in hs]

    ones2 = jnp.ones((c, 2), _BF16)
    ress = list(res0s)
    accs = [jnp.zeros((c,), _F32) for _ in xs]
    for dist_rhs, emb_rhs in ((dr0, er0), (dr1, er1), (dr2, er2)):
        lhss = [jnp.concatenate([r.astype(_BF16), ones2], axis=1) for r in ress]
        dists = [jnp.dot(l, dist_rhs[...], preferred_element_type=_F32)
                 for l in lhss]
        minvs = [jnp.min(d, axis=-1, keepdims=True) for d in dists]
        onehots = [(d == m).astype(_BF16) for d, m in zip(dists, minvs)]
        sels = [jax.lax.dot_general(o, emb_rhs[...], (((1,), (1,)), ((), ())),
                                    preferred_element_type=_F32)
                for o in onehots]
        embs = [(s[:, :32] + s[:, 32:64]) / s[:, 64:65] for s in sels]
        ress = [r - e for r, e in zip(ress, embs)]
        accs = [a + jnp.sum(r * r, axis=-1) for a, r in zip(accs, ress)]

    es = [r0 - r for r0, r in zip(res0s, ress)]
    hs = [jnp.maximum(_bdot(v, dw0[...]) + db0[0, :], 0.0) for v in es]
    hs = [jnp.maximum(_bdot(v, dw1[...]) + db1[0, :], 0.0) for v in hs]
    x_hats = [_bdot(v, dw2[...]) + db2[0, :] for v in hs]
    partial = sum(
        jnp.sum(jnp.sum((xh - v) ** 2, axis=-1) + (1.0 + BETA) * a)
        for xh, v, a in zip(x_hats, xs, accs))
    out_ref[...] = jnp.full((1, 1, 128), partial, _F32)


def _codebook_operands(cb):
    ct = cb.T.astype(_F32)
    k = ct.shape[1]
    cb2 = jnp.sum(ct * ct, axis=0, keepdims=True)
    cb2_hi = cb2.astype(_BF16)
    cb2_lo = (cb2 - cb2_hi.astype(_F32)).astype(_BF16)
    dist_rhs = jnp.concatenate(
        [(-2.0 * ct).astype(_BF16), cb2_hi, cb2_lo], axis=0)
    c_hi = ct.astype(_BF16)
    c_lo = (ct - c_hi.astype(_F32)).astype(_BF16)
    emb_rhs = jnp.concatenate(
        [c_hi, c_lo, jnp.ones((1, k), _BF16)], axis=0)
    return dist_rhs, emb_rhs


def kernel(x, gumbel_t, enc_w0, enc_b0, enc_w1, enc_b1, enc_w2, enc_b2,
           dec_w0, dec_b0, dec_w1, dec_b1, dec_w2, dec_b2,
           codebook0, codebook1, codebook2):
    b = x.shape[0]
    num_tiles = b // TILE
    biases = [jnp.reshape(v, (1, -1)) for v in
              (enc_b0, enc_b1, enc_b2, dec_b0, dec_b1, dec_b2)]
    ws = [w.astype(_BF16) for w in
          (enc_w0, enc_w1, enc_w2, dec_w0, dec_w1, dec_w2)]
    cb_ops = [_codebook_operands(cb) for cb in (codebook0, codebook1, codebook2)]

    def whole(a):
        return pl.BlockSpec(a.shape, lambda i: (0,) * a.ndim)

    ops = [ws[0], biases[0], ws[1], biases[1], ws[2], biases[2],
           ws[3], biases[3], ws[4], biases[4], ws[5], biases[5],
           cb_ops[0][0], cb_ops[1][0], cb_ops[2][0],
           cb_ops[0][1], cb_ops[1][1], cb_ops[2][1]]
    in_specs = [pl.BlockSpec((TILE, x.shape[1]), lambda i: (i, 0))]
    in_specs += [whole(a) for a in ops]

    partials = pl.pallas_call(
        _rqvae_tile,
        grid=(num_tiles,),
        in_specs=in_specs,
        out_specs=pl.BlockSpec((1, 1, 128), lambda i: (i, 0, 0)),
        out_shape=jax.ShapeDtypeStruct((num_tiles, 1, 128), _F32),
        compiler_params=pltpu.CompilerParams(
            dimension_semantics=("parallel",)),
    )(x, *ops)
    return jnp.sum(partials[:, 0, 0]) / b


# speedup vs baseline: 1.0075x; 1.0075x over previous
import jax
import jax.numpy as jnp
from jax.experimental import pallas as pl
from jax.experimental.pallas import tpu as pltpu

BETA = 0.25
TILE = 1024
NCHAIN = 2
_F32 = jnp.float32
_BF16 = jnp.bfloat16


def _bdot(a, b):
    return jnp.dot(a.astype(_BF16), b, preferred_element_type=_F32)


def _rqvae_tile(x_ref, ew0, eb0, ew1, eb1, ew2, eb2,
                dw0, db0, dw1, db1, dw2, db2,
                dr0, dr1, dr2, er0, er1, er2, out_ref):
    x = x_ref[...]
    t = x.shape[0]
    c = t // NCHAIN
    xs = [x[i * c:(i + 1) * c] for i in range(NCHAIN)]
    hs = [jnp.maximum(_bdot(v, ew0[...]) + eb0[0, :], 0.0) for v in xs]
    hs = [jnp.maximum(_bdot(v, ew1[...]) + eb1[0, :], 0.0) for v in hs]
    res0s = [_bdot(v, ew2[...]) + eb2[0, :] for v in hs]

    ones2 = jnp.ones((c, 2), _BF16)
    ress = list(res0s)
    accs = [jnp.zeros((c,), _F32) for _ in xs]
    for dist_rhs, emb_rhs in ((dr0, er0), (dr1, er1), (dr2, er2)):
        lhss = [jnp.concatenate([r.astype(_BF16), ones2], axis=1) for r in ress]
        dists = [jnp.dot(l, dist_rhs[...], preferred_element_type=_F32)
                 for l in lhss]
        minvs = [jnp.min(d, axis=-1, keepdims=True) for d in dists]
        onehots = [(d == m).astype(_BF16) for d, m in zip(dists, minvs)]
        sels = [jax.lax.dot_general(o, emb_rhs[...], (((1,), (1,)), ((), ())),
                                    preferred_element_type=_F32)
                for o in onehots]
        embs = [(s[:, :32] + s[:, 32:64]) / s[:, 64:65] for s in sels]
        ress = [r - e for r, e in zip(ress, embs)]
        accs = [a + jnp.sum(r * r, axis=-1) for a, r in zip(accs, ress)]

    es = [r0 - r for r0, r in zip(res0s, ress)]
    hs = [jnp.maximum(_bdot(v, dw0[...]) + db0[0, :], 0.0) for v in es]
    hs = [jnp.maximum(_bdot(v, dw1[...]) + db1[0, :], 0.0) for v in hs]
    x_hats = [_bdot(v, dw2[...]) + db2[0, :] for v in hs]
    partial = sum(
        jnp.sum(jnp.sum((xh - v) ** 2, axis=-1) + (1.0 + BETA) * a)
        for xh, v, a in zip(x_hats, xs, accs))
    out_ref[...] = jnp.full((1, 1, 128), partial, _F32)


def _codebook_operands(cb):
    ct = cb.T.astype(_F32)
    k = ct.shape[1]
    cb2 = jnp.sum(ct * ct, axis=0, keepdims=True)
    cb2_hi = cb2.astype(_BF16)
    cb2_lo = (cb2 - cb2_hi.astype(_F32)).astype(_BF16)
    dist_rhs = jnp.concatenate(
        [(-2.0 * ct).astype(_BF16), cb2_hi, cb2_lo], axis=0)
    c_hi = ct.astype(_BF16)
    c_lo = (ct - c_hi.astype(_F32)).astype(_BF16)
    emb_rhs = jnp.concatenate(
        [c_hi, c_lo, jnp.ones((1, k), _BF16)], axis=0)
    return dist_rhs, emb_rhs


def kernel(x, gumbel_t, enc_w0, enc_b0, enc_w1, enc_b1, enc_w2, enc_b2,
           dec_w0, dec_b0, dec_w1, dec_b1, dec_w2, dec_b2,
           codebook0, codebook1, codebook2):
    b = x.shape[0]
    num_tiles = b // TILE
    biases = [jnp.reshape(v, (1, -1)) for v in
              (enc_b0, enc_b1, enc_b2, dec_b0, dec_b1, dec_b2)]
    ws = [w.astype(_BF16) for w in
          (enc_w0, enc_w1, enc_w2, dec_w0, dec_w1, dec_w2)]
    cb_ops = [_codebook_operands(cb) for cb in (codebook0, codebook1, codebook2)]

    def whole(a):
        return pl.BlockSpec(a.shape, lambda i: (0,) * a.ndim)

    ops = [ws[0], biases[0], ws[1], biases[1], ws[2], biases[2],
           ws[3], biases[3], ws[4], biases[4], ws[5], biases[5],
           cb_ops[0][0], cb_ops[1][0], cb_ops[2][0],
           cb_ops[0][1], cb_ops[1][1], cb_ops[2][1]]
    in_specs = [pl.BlockSpec((TILE, x.shape[1]), lambda i: (i, 0))]
    in_specs += [whole(a) for a in ops]

    partials = pl.pallas_call(
        _rqvae_tile,
        grid=(num_tiles,),
        in_specs=in_specs,
        out_specs=pl.BlockSpec((1, 1, 128), lambda i: (i, 0, 0)),
        out_shape=jax.ShapeDtypeStruct((num_tiles, 1, 128), _F32),
        compiler_params=pltpu.CompilerParams(
            dimension_semantics=("parallel",)),
    )(x, *ops)
    return jnp.sum(partials[:, 0, 0]) / b
